# baseline (speedup 1.0000x reference)
import functools
import math

import jax
import jax.numpy as jnp
from jax import lax
from jax.experimental import pallas as pl
from jax.experimental.pallas import tpu as pltpu
from jax.experimental.pallas import tpu_sc as plsc

NC = 2
NS = 16
L = 16
NW = NC * NS

R = 4096
S = 200
D = 64
JB = S // 8
IB = R // 128
NG = 4
NCHUNK = S
NGROUP = NCHUNK // NG
SCALE = math.sqrt(D)

_mesh = plsc.VectorSubcoreMesh(core_axis_name="c", subcore_axis_name="s")


@functools.partial(
    pl.kernel,
    out_type=jax.ShapeDtypeStruct((S, 8, IB, 8, 128), jnp.float32),
    mesh=_mesh,
    scratch_types=[
        pltpu.VMEM((JB, 8, 128), jnp.int32),
        pltpu.VMEM((NG, 128, D), jnp.float32),
        pltpu.VMEM((NG, 8, 8, 128), jnp.float32),
    ]
    + [pltpu.SemaphoreType.DMA] * (2 * NG),
    compiler_params=pltpu.CompilerParams(
        use_tc_tiling_on_sc=False, needs_layout_passes=False
    ),
)
def _embed(xt_hbm, table_hbm, out_hbm, idx_v, g_v, o_v, *sems):
    gsem, osem = sems[:NG], sems[NG:]
    wid = lax.axis_index("s") * NC + lax.axis_index("c")

    def stage(jb, carry):
        pltpu.sync_copy(xt_hbm.at[jb, wid], idx_v.at[jb])
        return carry

    lax.fori_loop(0, JB, stage, 0)

    rows0 = lax.iota(jnp.int32, L)

    def start_gather(b, j):
        pltpu.async_copy(
            table_hbm.at[idx_v.at[j // 8, j % 8]], g_v.at[b], gsem[b]
        )

    for b in range(NG):
        start_gather(b, b)

    def group(g, carry):
        j0 = NG * g
        for b in range(NG):
            j = j0 + b
            pltpu.make_async_copy(
                table_hbm.at[idx_v.at[0, 0]], g_v.at[b], gsem[b]
            ).wait()

            @pl.when(g > 0)
            def _():
                for k8 in range(8):
                    pltpu.make_async_copy(
                        o_v.at[b, k8], out_hbm.at[0, k8, 0], osem[b]
                    ).wait()

            def krow_body(krow, c2, b=b):
                colv = jnp.full((L,), krow, dtype=jnp.int32)
                k8 = krow // 8
                rt = krow % 8
                for c0 in range(8):
                    v = plsc.load_gather(g_v.at[b], [rows0 + c0 * L, colv])
                    o_v[b, k8, rt, pl.ds(c0 * L, L)] = v * SCALE
                return c2

            lax.fori_loop(0, 64, krow_body, 0)

            for k8 in range(8):
                pltpu.async_copy(
                    o_v.at[b, k8], out_hbm.at[j, k8, wid], osem[b]
                )

            @pl.when(g < NGROUP - 1)
            def _():
                start_gather(b, j + NG)
        return carry

    lax.fori_loop(0, NGROUP, group, 0)

    for b in range(NG):
        for k8 in range(8):
            pltpu.make_async_copy(
                o_v.at[b, k8], out_hbm.at[0, k8, 0], osem[b]
            ).wait()


def kernel(x, table):
    xt = x.T.reshape(JB, 8, IB, 128).transpose(0, 2, 1, 3)
    out5 = _embed(xt, table)
    return out5.transpose(2, 4, 0, 1, 3).reshape(R, S, D)
